# baseline (speedup 1.0000x reference)
import dataclasses
import functools

import jax
import jax.numpy as jnp
from jax import lax
from jax.experimental import pallas as pl
from jax.experimental.pallas import tpu as pltpu
from jax.experimental.pallas import tpu_sc as plsc

DZ = 2.0
DTNEW = 0.001
NTNEW = 2000
NWAV = 101
ND = 2000
NV = 2001
HALO = NWAV - 1
PW = 2304
TT = 512
KW = TT + 128
NTILES = 4
LANES = 16
NWORKERS = 32
CSBLK = 128
MAGIC = float(1.5 * 2**23)


def _sc_fused(dt_t, refl_t, batch):
    mesh = plsc.VectorSubcoreMesh(core_axis_name="c", subcore_axis_name="s")
    cp = pltpu.CompilerParams()
    if "needs_layout_passes" in pltpu.CompilerParams.__dataclass_fields__:
        cp = dataclasses.replace(cp, needs_layout_passes=False)
    rows_per_worker = batch // NWORKERS
    groups = rows_per_worker // LANES
    nblocks = [CSBLK] * (ND // CSBLK) + ([ND % CSBLK] if ND % CSBLK else [])

    gsz = ND * LANES

    @functools.partial(
        pl.kernel,
        out_type=jax.ShapeDtypeStruct((batch, PW), jnp.float32),
        mesh=mesh,
        compiler_params=cp,
        scratch_types=[
            pltpu.VMEM((gsz,), jnp.float32),
            pltpu.VMEM((gsz,), jnp.float32),
            pltpu.VMEM((LANES, PW), jnp.float32),
        ],
    )
    def k(dt_hbm, refl_hbm, p_hbm, dtbuf, reflbuf, ptime):
        wid = lax.axis_index("s") * 2 + lax.axis_index("c")
        lanes = lax.broadcasted_iota(jnp.int32, (LANES,), 0)
        zeros = jnp.zeros((LANES,), jnp.float32)

        @pl.loop(0, groups)
        def _group(g):
            gidx = wid * groups + g
            row0 = gidx * LANES
            pltpu.sync_copy(dt_hbm.at[pl.ds(gidx * gsz, gsz)], dtbuf)
            pltpu.sync_copy(refl_hbm.at[pl.ds(gidx * gsz, gsz)], reflbuf)

            for r in range(LANES):
                @pl.loop(0, PW, step=LANES)
                def _zero(cc, r=r):
                    ptime.at[r, pl.ds(cc, LANES)][...] = zeros

            carry = (
                zeros,
                zeros,
                jnp.full((LANES,), 2**20, jnp.int32),
                zeros,
            )

            def step(i, carry):
                w, c, prev_idx, prev_refl = carry
                w = w + dtbuf[pl.ds(i * LANES, LANES)]
                x = (c + w) / DTNEW
                idx = ((x + MAGIC) - MAGIC).astype(jnp.int32)
                refl = reflbuf[pl.ds(i * LANES, LANES)]
                m = (prev_idx != idx) & (prev_idx < NTNEW)
                plsc.store_scatter(
                    ptime, [lanes, prev_idx + HALO], prev_refl, mask=m)
                return (w, c, idx, refl)

            i0 = 0
            for nb in nblocks:
                carry = lax.fori_loop(
                    i0, i0 + nb, step, carry, unroll=8)
                w, c, prev_idx, prev_refl = carry
                carry = (zeros, c + w, prev_idx, prev_refl)
                i0 += nb

            _, _, prev_idx, prev_refl = carry
            m = prev_idx < NTNEW
            plsc.store_scatter(
                ptime, [lanes, prev_idx + HALO], prev_refl, mask=m)

            pltpu.sync_copy(ptime, p_hbm.at[pl.ds(row0, LANES), :])

    return k(dt_t, refl_t)


def _conv_body(p_ref, w_ref, o_ref):
    w = w_ref[...]
    for j in range(NTILES):
        o_ref[:, j * TT:(j + 1) * TT] = jnp.dot(
            p_ref[:, j * TT:j * TT + KW], w,
            preferred_element_type=jnp.float32)


def _tc_conv(p, wband, batch):
    bm = 256
    return pl.pallas_call(
        _conv_body,
        grid=(batch // bm,),
        in_specs=[
            pl.BlockSpec((bm, PW), lambda i: (i, 0)),
            pl.BlockSpec((KW, TT), lambda i: (0, 0)),
        ],
        out_specs=pl.BlockSpec((bm, NTILES * TT), lambda i: (i, 0)),
        out_shape=jax.ShapeDtypeStruct((batch, NTILES * TT), jnp.float32),
    )(p, wband)


def kernel(vmodel, wavelet):
    batch = vmodel.shape[0]
    v0 = vmodel[:, :-1]
    v1 = vmodel[:, 1:]
    dt = DZ / v0
    refl = (v1 - v0) / (v1 + v0)
    ngroups = batch // LANES
    dt_t = dt.reshape(ngroups, LANES, ND).swapaxes(1, 2).reshape(-1)
    refl_t = refl.reshape(ngroups, LANES, ND).swapaxes(1, 2).reshape(-1)
    p = _sc_fused(dt_t, refl_t, batch)

    k = lax.broadcasted_iota(jnp.int32, (KW, TT), 0)
    u = lax.broadcasted_iota(jnp.int32, (KW, TT), 1)
    j = u - k + HALO
    band = (j >= 0) & (j <= HALO)
    wband = jnp.where(band, jnp.take(wavelet, jnp.clip(j, 0, HALO)), 0.0)
    wband = wband.astype(jnp.float32)

    out = _tc_conv(p, wband, batch)
    return out[:, :NTNEW]


# speedup vs baseline: 1.1100x; 1.1100x over previous
import dataclasses
import functools

import jax
import jax.numpy as jnp
from jax import lax
from jax.experimental import pallas as pl
from jax.experimental.pallas import tpu as pltpu
from jax.experimental.pallas import tpu_sc as plsc

DZ = 2.0
DTNEW = 0.001
NTNEW = 2000
NWAV = 101
ND = 2000
NV = 2001
HALO = NWAV - 1
PW = 2304
TT = 512
KW = TT + 128
NTILES = 4
LANES = 16
NWORKERS = 32
CSBLK = 128
MAGIC = float(1.5 * 2**23)


def _sc_fused(dt_t, refl_t, batch):
    mesh = plsc.VectorSubcoreMesh(core_axis_name="c", subcore_axis_name="s")
    cp = pltpu.CompilerParams()
    if "needs_layout_passes" in pltpu.CompilerParams.__dataclass_fields__:
        cp = dataclasses.replace(cp, needs_layout_passes=False)
    rows_per_worker = batch // NWORKERS
    groups = rows_per_worker // LANES
    nblocks = [CSBLK] * (ND // CSBLK) + ([ND % CSBLK] if ND % CSBLK else [])

    @functools.partial(
        pl.kernel,
        out_type=jax.ShapeDtypeStruct((batch, PW), jnp.float32),
        mesh=mesh,
        compiler_params=cp,
        scratch_types=[
            pltpu.VMEM((LANES, ND), jnp.float32),
            pltpu.VMEM((LANES, ND), jnp.float32),
            pltpu.VMEM((LANES, PW), jnp.float32),
        ],
    )
    def k(dt_hbm, refl_hbm, p_hbm, dtbuf, reflbuf, ptime):
        wid = lax.axis_index("s") * 2 + lax.axis_index("c")
        lanes = lax.broadcasted_iota(jnp.int32, (LANES,), 0)
        zeros = jnp.zeros((LANES,), jnp.float32)

        @pl.loop(0, groups)
        def _group(g):
            row0 = (wid * groups + g) * LANES
            pltpu.sync_copy(dt_hbm.at[pl.ds(row0, LANES), :], dtbuf)
            pltpu.sync_copy(refl_hbm.at[pl.ds(row0, LANES), :], reflbuf)

            for r in range(LANES):
                @pl.loop(0, PW, step=LANES)
                def _zero(cc, r=r):
                    ptime.at[r, pl.ds(cc, LANES)][...] = zeros

            carry = (
                jnp.zeros((LANES,), jnp.int32),
                zeros,
                zeros,
                jnp.full((LANES,), 2**20, jnp.int32),
                zeros,
            )

            def step(i, carry):
                colv, w, c, prev_idx, prev_refl = carry
                w = w + plsc.load_gather(dtbuf, [lanes, colv])
                x = (c + w) / DTNEW
                idx = ((x + MAGIC) - MAGIC).astype(jnp.int32)
                refl = plsc.load_gather(reflbuf, [lanes, colv])
                m = (prev_idx != idx) & (prev_idx < NTNEW)
                plsc.store_scatter(
                    ptime, [lanes, prev_idx + HALO], prev_refl, mask=m)
                return (colv + 1, w, c, idx, refl)

            i0 = 0
            for nb in nblocks:
                carry = lax.fori_loop(
                    i0, i0 + nb, step, carry, unroll=8)
                colv, w, c, prev_idx, prev_refl = carry
                carry = (colv, zeros, c + w, prev_idx, prev_refl)
                i0 += nb

            _, _, _, prev_idx, prev_refl = carry
            m = prev_idx < NTNEW
            plsc.store_scatter(
                ptime, [lanes, prev_idx + HALO], prev_refl, mask=m)

            pltpu.sync_copy(ptime, p_hbm.at[pl.ds(row0, LANES), :])

    return k(dt_t, refl_t)


def _conv_body(p_ref, w_ref, o_ref):
    w = w_ref[...]
    for j in range(NTILES):
        o_ref[j * TT:(j + 1) * TT, :] = lax.dot_general(
            w, p_ref[:, j * TT:j * TT + KW],
            dimension_numbers=(((0,), (1,)), ((), ())),
            preferred_element_type=jnp.float32)


def _tc_conv(p, wband, batch):
    bm = 256
    return pl.pallas_call(
        _conv_body,
        grid=(batch // bm,),
        in_specs=[
            pl.BlockSpec((bm, PW), lambda i: (i, 0)),
            pl.BlockSpec((KW, TT), lambda i: (0, 0)),
        ],
        out_specs=pl.BlockSpec((NTILES * TT, bm), lambda i: (0, i)),
        out_shape=jax.ShapeDtypeStruct((NTILES * TT, batch), jnp.float32),
    )(p, wband)


def kernel(vmodel, wavelet):
    batch = vmodel.shape[0]
    v0 = vmodel[:, :-1]
    v1 = vmodel[:, 1:]
    dt = DZ / v0
    refl = (v1 - v0) / (v1 + v0)
    p = _sc_fused(dt, refl, batch)

    k = lax.broadcasted_iota(jnp.int32, (KW, TT), 0)
    u = lax.broadcasted_iota(jnp.int32, (KW, TT), 1)
    j = u - k + HALO
    band = (j >= 0) & (j <= HALO)
    wband = jnp.where(band, jnp.take(wavelet, jnp.clip(j, 0, HALO)), 0.0)
    wband = wband.astype(jnp.float32)

    out_t = _tc_conv(p, wband, batch)
    return out_t[:NTNEW, :].T


# speedup vs baseline: 3.7781x; 3.4038x over previous
import dataclasses
import functools

import jax
import jax.numpy as jnp
from jax import lax
from jax.experimental import pallas as pl
from jax.experimental.pallas import tpu as pltpu
from jax.experimental.pallas import tpu_sc as plsc

DZ = 2.0
DTNEW = 0.001
NTNEW = 2000
NWAV = 101
ND = 2000
NV = 2001
HALO = NWAV - 1
PW = 2304
TT = 512
KW = TT + 128
NTILES = 4
LANES = 16
NWORKERS = 32
CSBLK = 128
MAGIC = float(1.5 * 2**23)


def _sc_fused(dt_t, refl_t, batch):
    mesh = plsc.VectorSubcoreMesh(core_axis_name="c", subcore_axis_name="s")
    cp = pltpu.CompilerParams()
    if "needs_layout_passes" in pltpu.CompilerParams.__dataclass_fields__:
        cp = dataclasses.replace(cp, needs_layout_passes=False)
    rows_per_worker = batch // NWORKERS
    groups = rows_per_worker // LANES
    nblocks = [CSBLK] * (ND // CSBLK) + ([ND % CSBLK] if ND % CSBLK else [])

    @functools.partial(
        pl.kernel,
        out_type=jax.ShapeDtypeStruct((batch, PW), jnp.float32),
        mesh=mesh,
        compiler_params=cp,
        scratch_types=[
            pltpu.VMEM((LANES, ND), jnp.float32),
            pltpu.VMEM((LANES, ND), jnp.float32),
            pltpu.VMEM((LANES, PW), jnp.float32),
        ],
    )
    def k(dt_hbm, refl_hbm, p_hbm, dtbuf, reflbuf, ptime):
        wid = lax.axis_index("s") * 2 + lax.axis_index("c")
        lanes = lax.broadcasted_iota(jnp.int32, (LANES,), 0)
        zeros = jnp.zeros((LANES,), jnp.float32)

        @pl.loop(0, groups)
        def _group(g):
            row0 = (wid * groups + g) * LANES
            pltpu.sync_copy(dt_hbm.at[pl.ds(row0, LANES), :], dtbuf)
            pltpu.sync_copy(refl_hbm.at[pl.ds(row0, LANES), :], reflbuf)

            for r in range(LANES):
                @pl.loop(0, PW, step=LANES)
                def _zero(cc, r=r):
                    ptime.at[r, pl.ds(cc, LANES)][...] = zeros

            carry = (
                jnp.zeros((LANES,), jnp.int32),
                zeros,
                zeros,
                jnp.full((LANES,), 2**20, jnp.int32),
                zeros,
            )

            def step(i, carry):
                colv, w, c, prev_idx, prev_refl = carry
                w = w + plsc.load_gather(dtbuf, [lanes, colv])
                x = (c + w) / DTNEW
                idx = ((x + MAGIC) - MAGIC).astype(jnp.int32)
                refl = plsc.load_gather(reflbuf, [lanes, colv])
                m = (prev_idx != idx) & (prev_idx < NTNEW)
                plsc.store_scatter(
                    ptime, [lanes, prev_idx + HALO], prev_refl, mask=m)
                return (colv + 1, w, c, idx, refl)

            i0 = 0
            for nb in nblocks:
                carry = lax.fori_loop(
                    i0, i0 + nb, step, carry, unroll=8)
                colv, w, c, prev_idx, prev_refl = carry
                carry = (colv, zeros, c + w, prev_idx, prev_refl)
                i0 += nb

            _, _, _, prev_idx, prev_refl = carry
            m = prev_idx < NTNEW
            plsc.store_scatter(
                ptime, [lanes, prev_idx + HALO], prev_refl, mask=m)

            pltpu.sync_copy(ptime, p_hbm.at[pl.ds(row0, LANES), :])

    return k(dt_t, refl_t)


def _conv_body(p_ref, w_ref, o_ref):
    w = w_ref[...]
    for j in range(NTILES):
        o_ref[j * TT:(j + 1) * TT, :] = lax.dot_general(
            w, p_ref[:, j * TT:j * TT + KW],
            dimension_numbers=(((0,), (1,)), ((), ())),
            preferred_element_type=jnp.float32)


def _tc_conv(p, wband, batch):
    bm = 256
    return pl.pallas_call(
        _conv_body,
        grid=(batch // bm,),
        in_specs=[
            pl.BlockSpec((bm, PW), lambda i: (i, 0)),
            pl.BlockSpec((KW, TT), lambda i: (0, 0)),
        ],
        out_specs=pl.BlockSpec((NTILES * TT, bm), lambda i: (0, i)),
        out_shape=jax.ShapeDtypeStruct((NTILES * TT, batch), jnp.float32),
    )(p, wband)


def kernel(vmodel, wavelet):
    batch = vmodel.shape[0]
    v0 = vmodel[:, :-1]
    v1 = vmodel[:, 1:]
    dt = DZ / v0
    refl = (v1 - v0) / (v1 + v0)
    p = _sc_fused(dt, refl, batch)

    circ = KW + TT
    v = jnp.concatenate([
        wavelet[HALO:],
        jnp.zeros((circ - NWAV,), jnp.float32),
        wavelet[:HALO],
    ]).astype(jnp.float32)
    wband = jnp.tile(v, KW)[: KW * (circ - 1)].reshape(KW, circ - 1)[:, :TT]

    out_t = _tc_conv(p, wband, batch)
    return out_t[:NTNEW, :].T
